# baseline (speedup 1.0000x reference)
import jax
import jax.numpy as jnp
from jax.experimental import pallas as pl
from jax.experimental.pallas import tpu as pltpu

_B = 8
_H = 16
_W = 512
_OVL = 4
_NVQ = 8
_K = 32
_CBS = 1024
_T = _W // _OVL
_C = 128
_HALF = 64
_TCH = 32
_NT = _T // _TCH
_WCH = _OVL * _TCH

_PREC = jax.lax.Precision.DEFAULT


def _vq_kernel(z_ref, dwe_ref, dwo_ref, uwe_ref, uwo_ref, cb_ref,
               out_ref, codes_ref, loss_ref, en_scr, ee_scr):
    t = pl.program_id(0)
    rows_all = _B * _WCH
    rows = _B * _TCH

    @pl.when(t == 0)
    def _prep():
        for m in range(_NVQ):
            e = cb_ref[m]
            en = e * (1.0 / (jnp.sqrt(jnp.sum(e * e, axis=1, keepdims=True))
                             + 1e-8))
            en_scr[m] = en
            ee_scr[m] = jnp.sum(en * en, axis=1).reshape(1, _CBS)

    acc_e = jnp.zeros((rows_all, _C), jnp.float32)
    acc_o = jnp.zeros((rows_all, _C), jnp.float32)
    for h in range(_H):
        zt = z_ref[:, h, :, :].reshape(rows_all, _C)
        acc_e = acc_e + jnp.dot(zt[:, :_HALF], dwe_ref[h],
                                preferred_element_type=jnp.float32,
                                precision=_PREC)
        acc_o = acc_o + jnp.dot(zt[:, _HALF:], dwo_ref[h],
                                preferred_element_type=jnp.float32,
                                precision=_PREC)

    acc_e4 = acc_e.reshape(_B, _TCH, _OVL, _C)
    acc_o4 = acc_o.reshape(_B, _TCH, _OVL, _C)
    loss_acc = jnp.float32(0.0)
    zqe_parts, zqo_parts = [], []
    for o in range(_OVL):
        for s in range(2):
            m = 2 * o + s
            accp = acc_o4 if s else acc_e4
            z_s = accp[:, :, o, _K * o:_K * (o + 1)].reshape(rows, _K)
            zn = z_s * (1.0 / (jnp.sqrt(jnp.sum(z_s * z_s, axis=1,
                                                keepdims=True)) + 1e-8))
            en = en_scr[m]
            ee = ee_scr[m]
            sim = jnp.dot(zn, en.T, preferred_element_type=jnp.float32,
                          precision=_PREC)
            zz = jnp.sum(zn * zn, axis=1, keepdims=True)
            d = zz - 2.0 * sim + ee
            code = jnp.argmin(d, axis=1).astype(jnp.int32)
            idx = jax.lax.broadcasted_iota(jnp.int32, d.shape, 1)
            codes_ref[0, m] = code.reshape(_B, _TCH)
            onehot = (code[:, None] == idx).astype(jnp.float32)
            zq = jnp.dot(onehot, en, preferred_element_type=jnp.float32,
                         precision=_PREC)
            loss_acc = loss_acc + jnp.sum((zq - z_s) ** 2)
            pieces = []
            if o:
                pieces.append(jnp.zeros((rows, _K * o), jnp.float32))
            pieces.append(zq)
            if o < _OVL - 1:
                pieces.append(jnp.zeros((rows, _C - _K * (o + 1)), jnp.float32))
            zq_full = jnp.concatenate(pieces, axis=1).reshape(
                _B, _TCH, 1, _C)
            (zqo_parts if s else zqe_parts).append(zq_full)
    zq_e = jnp.concatenate(zqe_parts, axis=2).reshape(rows_all, _C)
    zq_o = jnp.concatenate(zqo_parts, axis=2).reshape(rows_all, _C)

    for h in range(_H):
        lo = jnp.dot(zq_e, uwe_ref[h], preferred_element_type=jnp.float32,
                     precision=_PREC)
        hi = jnp.dot(zq_o, uwo_ref[h], preferred_element_type=jnp.float32,
                     precision=_PREC)
        out_ref[:, h, :, :] = jnp.concatenate([lo, hi], axis=1).reshape(
            _B, _WCH, _C)

    @pl.when(t == 0)
    def _init():
        loss_ref[...] = jnp.zeros((1, 1), jnp.float32)

    loss_ref[...] += loss_acc.reshape(1, 1)


def _pack_weights(down_Ws, up_Ws):
    def dpack(w):
        a = w.reshape(_OVL, _K, _HALF, _H).transpose(3, 2, 0, 1)
        return a.reshape(_H, _HALF, _OVL * _K)

    def upack(w):
        u = w.reshape(_OVL, _HALF, _H, _K).transpose(2, 0, 3, 1)
        return u.reshape(_H, _OVL * _K, _HALF)

    return (dpack(down_Ws[0::2]), dpack(down_Ws[1::2]),
            upack(up_Ws[0::2]), upack(up_Ws[1::2]))


def kernel(z_e, down_Ws, up_Ws, codebooks):
    ze = z_e.reshape(_B, _H, _W, _C)
    dwe, dwo, uwe, uwo = _pack_weights(down_Ws, up_Ws)

    out, codes_raw, loss_raw = pl.pallas_call(
        _vq_kernel,
        grid=(_NT,),
        in_specs=[
            pl.BlockSpec((_B, _H, _WCH, _C), lambda t: (0, 0, t, 0)),
            pl.BlockSpec((_H, _HALF, _C), lambda t: (0, 0, 0)),
            pl.BlockSpec((_H, _HALF, _C), lambda t: (0, 0, 0)),
            pl.BlockSpec((_H, _C, _HALF), lambda t: (0, 0, 0)),
            pl.BlockSpec((_H, _C, _HALF), lambda t: (0, 0, 0)),
            pl.BlockSpec((_NVQ, _CBS, _K), lambda t: (0, 0, 0)),
        ],
        out_specs=[
            pl.BlockSpec((_B, _H, _WCH, _C), lambda t: (0, 0, t, 0)),
            pl.BlockSpec((1, _NVQ, _B, _TCH), lambda t: (t, 0, 0, 0)),
            pl.BlockSpec((1, 1), lambda t: (0, 0)),
        ],
        out_shape=[
            jax.ShapeDtypeStruct((_B, _H, _W, _C), jnp.float32),
            jax.ShapeDtypeStruct((_NT, _NVQ, _B, _TCH), jnp.int32),
            jax.ShapeDtypeStruct((1, 1), jnp.float32),
        ],
        scratch_shapes=[
            pltpu.VMEM((_NVQ, _CBS, _K), jnp.float32),
            pltpu.VMEM((_NVQ, 1, _CBS), jnp.float32),
        ],
    )(ze, dwe, dwo, uwe, uwo, codebooks)

    z_q = out.reshape(_B, _H * _W, _C)
    codes = codes_raw.transpose(2, 1, 0, 3).reshape(_B, _NVQ, _T)
    loss = loss_raw[0, 0] / jnp.float32(_NVQ * _B * _T * _K)
    return z_q, codes, loss, loss


# speedup vs baseline: 1.3526x; 1.3526x over previous
import jax
import jax.numpy as jnp
from jax.experimental import pallas as pl
from jax.experimental.pallas import tpu as pltpu

_B = 8
_H = 16
_W = 512
_OVL = 4
_NVQ = 8
_K = 32
_CBS = 1024
_T = _W // _OVL
_C = 128
_HALF = 64
_TCH = 32
_NT = _T // _TCH
_WCH = _OVL * _TCH

_PREC = jax.lax.Precision.DEFAULT


def _vq_kernel(z_ref, dwe_ref, dwo_ref, uwe_ref, uwo_ref, cb_ref,
               out_ref, codes_ref, loss_ref, en_scr, ee_scr):
    t = pl.program_id(0)
    rows_all = _B * _WCH
    rows = _B * _TCH

    @pl.when(t == 0)
    def _prep():
        for m in range(_NVQ):
            e = cb_ref[m]
            en = e * (1.0 / (jnp.sqrt(jnp.sum(e * e, axis=1, keepdims=True))
                             + 1e-8))
            en_scr[m] = en
            ee_scr[m] = jnp.sum(en * en, axis=1).reshape(1, _CBS)

    acc_e = jnp.zeros((rows_all, _C), jnp.float32)
    acc_o = jnp.zeros((rows_all, _C), jnp.float32)
    for h in range(_H):
        zt = z_ref[:, h, :, :].reshape(rows_all, _C)
        acc_e = acc_e + jnp.dot(zt[:, :_HALF], dwe_ref[h],
                                preferred_element_type=jnp.float32,
                                precision=_PREC)
        acc_o = acc_o + jnp.dot(zt[:, _HALF:], dwo_ref[h],
                                preferred_element_type=jnp.float32,
                                precision=_PREC)

    acc_e4 = acc_e.reshape(_B, _TCH, _OVL, _C)
    acc_o4 = acc_o.reshape(_B, _TCH, _OVL, _C)
    loss_acc = jnp.float32(0.0)
    zqe_parts, zqo_parts = [], []
    for o in range(_OVL):
        for s in range(2):
            m = 2 * o + s
            accp = acc_o4 if s else acc_e4
            z_s = accp[:, :, o, _K * o:_K * (o + 1)].reshape(rows, _K)
            zn = z_s * (1.0 / (jnp.sqrt(jnp.sum(z_s * z_s, axis=1,
                                                keepdims=True)) + 1e-8))
            en = en_scr[m]
            ee = ee_scr[m]
            sim = jnp.dot(zn, en.T, preferred_element_type=jnp.float32,
                          precision=_PREC)
            zz = jnp.sum(zn * zn, axis=1, keepdims=True)
            d = zz - 2.0 * sim + ee
            dmin = jnp.min(d, axis=1, keepdims=True)
            idx = jax.lax.broadcasted_iota(jnp.int32, d.shape, 1)
            code = jnp.min(jnp.where(d <= dmin, idx, _CBS), axis=1)
            codes_ref[0, m] = code.reshape(_B, _TCH)
            onehot = (code[:, None] == idx).astype(jnp.float32)
            zq = jnp.dot(onehot, en, preferred_element_type=jnp.float32,
                         precision=_PREC)
            loss_acc = loss_acc + jnp.sum((zq - z_s) ** 2)
            pieces = []
            if o:
                pieces.append(jnp.zeros((rows, _K * o), jnp.float32))
            pieces.append(zq)
            if o < _OVL - 1:
                pieces.append(jnp.zeros((rows, _C - _K * (o + 1)), jnp.float32))
            zq_full = jnp.concatenate(pieces, axis=1).reshape(
                _B, _TCH, 1, _C)
            (zqo_parts if s else zqe_parts).append(zq_full)
    zq_e = jnp.concatenate(zqe_parts, axis=2).reshape(rows_all, _C)
    zq_o = jnp.concatenate(zqo_parts, axis=2).reshape(rows_all, _C)

    for h in range(_H):
        lo = jnp.dot(zq_e, uwe_ref[h], preferred_element_type=jnp.float32,
                     precision=_PREC)
        hi = jnp.dot(zq_o, uwo_ref[h], preferred_element_type=jnp.float32,
                     precision=_PREC)
        out_ref[:, h, :, :] = jnp.concatenate([lo, hi], axis=1).reshape(
            _B, _WCH, _C)

    @pl.when(t == 0)
    def _init():
        loss_ref[...] = jnp.zeros((1, 1), jnp.float32)

    loss_ref[...] += loss_acc.reshape(1, 1)


def _pack_weights(down_Ws, up_Ws):
    def dpack(w):
        a = w.reshape(_OVL, _K, _HALF, _H).transpose(3, 2, 0, 1)
        return a.reshape(_H, _HALF, _OVL * _K)

    def upack(w):
        u = w.reshape(_OVL, _HALF, _H, _K).transpose(2, 0, 3, 1)
        return u.reshape(_H, _OVL * _K, _HALF)

    return (dpack(down_Ws[0::2]), dpack(down_Ws[1::2]),
            upack(up_Ws[0::2]), upack(up_Ws[1::2]))


def kernel(z_e, down_Ws, up_Ws, codebooks):
    ze = z_e.reshape(_B, _H, _W, _C)
    dwe, dwo, uwe, uwo = _pack_weights(down_Ws, up_Ws)

    out, codes_raw, loss_raw = pl.pallas_call(
        _vq_kernel,
        grid=(_NT,),
        in_specs=[
            pl.BlockSpec((_B, _H, _WCH, _C), lambda t: (0, 0, t, 0)),
            pl.BlockSpec((_H, _HALF, _C), lambda t: (0, 0, 0)),
            pl.BlockSpec((_H, _HALF, _C), lambda t: (0, 0, 0)),
            pl.BlockSpec((_H, _C, _HALF), lambda t: (0, 0, 0)),
            pl.BlockSpec((_H, _C, _HALF), lambda t: (0, 0, 0)),
            pl.BlockSpec((_NVQ, _CBS, _K), lambda t: (0, 0, 0)),
        ],
        out_specs=[
            pl.BlockSpec((_B, _H, _WCH, _C), lambda t: (0, 0, t, 0)),
            pl.BlockSpec((1, _NVQ, _B, _TCH), lambda t: (t, 0, 0, 0)),
            pl.BlockSpec((1, 1), lambda t: (0, 0)),
        ],
        out_shape=[
            jax.ShapeDtypeStruct((_B, _H, _W, _C), jnp.float32),
            jax.ShapeDtypeStruct((_NT, _NVQ, _B, _TCH), jnp.int32),
            jax.ShapeDtypeStruct((1, 1), jnp.float32),
        ],
        scratch_shapes=[
            pltpu.VMEM((_NVQ, _CBS, _K), jnp.float32),
            pltpu.VMEM((_NVQ, 1, _CBS), jnp.float32),
        ],
    )(ze, dwe, dwo, uwe, uwo, codebooks)

    z_q = out.reshape(_B, _H * _W, _C)
    codes = codes_raw.transpose(2, 1, 0, 3).reshape(_B, _NVQ, _T)
    loss = loss_raw[0, 0] / jnp.float32(_NVQ * _B * _T * _K)
    return z_q, codes, loss, loss
